# Initial kernel scaffold; baseline (speedup 1.0000x reference)
import functools

import jax
import jax.numpy as jnp
from jax import lax
from jax.experimental import pallas as pl
from jax.experimental.pallas import tpu as pltpu
from jax.experimental.pallas import tpu_sc as plsc

N = 10000
E = 320000
D = 128

NC = 2
NS = 16
NW = NC * NS
EPW = E // NW
CHUNK = 80
NCH = EPW // CHUNK
RPT = N // NS

_mesh = plsc.VectorSubcoreMesh(core_axis_name="c", subcore_axis_name="s")


@functools.partial(
    pl.kernel,
    out_type=jax.ShapeDtypeStruct((E, D), jnp.float32),
    mesh=_mesh,
    scratch_types=[
        pltpu.VMEM((NCH, CHUNK), jnp.int32),
        pltpu.VMEM((NCH, CHUNK), jnp.int32),
        pltpu.VMEM((CHUNK, D), jnp.float32),
        pltpu.SemaphoreType.DMA,
    ],
)
def _sc_gather_sum(pa_hbm, pb_hbm, ia_hbm, ib_hbm, out_hbm, ia_v, ib_v, rows_v, sem):
    wid = lax.axis_index("c") * NS + lax.axis_index("s")
    base = wid * EPW
    pltpu.sync_copy(ia_hbm.at[wid], ia_v)
    pltpu.sync_copy(ib_hbm.at[wid], ib_v)

    def step(j, carry):
        pltpu.async_copy(pa_hbm.at[ia_v.at[j]], rows_v, sem).wait()
        pltpu.async_copy(pb_hbm.at[ib_v.at[j]], rows_v, sem, add=True).wait()
        pltpu.sync_copy(rows_v, out_hbm.at[pl.ds(base + j * CHUNK, CHUNK)])
        return carry

    lax.fori_loop(0, NCH, step, 0)


@functools.partial(
    pl.kernel,
    out_type=jax.ShapeDtypeStruct((NC, N, D), jnp.float32),
    mesh=_mesh,
    scratch_types=[
        pltpu.VMEM((NCH, CHUNK), jnp.int32),
        pltpu.VMEM((CHUNK, D), jnp.float32),
        pltpu.VMEM_SHARED((N, D), jnp.float32),
    ],
)
def _sc_scatter_add(m_hbm, idx_hbm, zeros_hbm, out_hbm, idx_v, rows_v, acc_sh):
    cid = lax.axis_index("c")
    sid = lax.axis_index("s")
    wid = cid * NS + sid
    base = wid * EPW
    pltpu.sync_copy(zeros_hbm.at[pl.ds(sid * RPT, RPT)],
                    acc_sh.at[pl.ds(sid * RPT, RPT)])
    pltpu.sync_copy(idx_hbm.at[wid], idx_v)
    plsc.subcore_barrier()

    def step(j, carry):
        pltpu.sync_copy(m_hbm.at[pl.ds(base + j * CHUNK, CHUNK)], rows_v)
        pltpu.sync_copy(rows_v, acc_sh.at[idx_v.at[j]], add=True)
        return carry

    lax.fori_loop(0, NCH, step, 0)
    plsc.subcore_barrier()
    pltpu.sync_copy(acc_sh.at[pl.ds(sid * RPT, RPT)],
                    out_hbm.at[cid, pl.ds(sid * RPT, RPT)])


def _dot(a, b):
    return jnp.dot(a, b, preferred_element_type=jnp.float32)


def _mlp_tail(h, w1, b1, w2, b2, gam, bet):
    h = jnp.maximum(_dot(h, w1) + b1, 0.0)
    h = _dot(h, w2) + b2
    mu = jnp.mean(h, axis=-1, keepdims=True)
    var = jnp.mean((h - mu) ** 2, axis=-1, keepdims=True)
    h = (h - mu) * lax.rsqrt(var + 1e-5)
    return h * gam + bet


def _edge_mlp_body(g_ref, e_ref, wc, b0, w1, b1, w2, b2, gam, bet, o_ref):
    h = jnp.maximum(g_ref[...] + _dot(e_ref[...], wc[...]) + b0[...], 0.0)
    o_ref[...] = _mlp_tail(h, w1[...], b1[...], w2[...], b2[...], gam[...], bet[...])


def _u_mlp_body(g_ref, e_ref, wc, b0, w1, b1, w2, b2, gam, bet, o_ref):
    ev = e_ref[...]
    h = jnp.maximum(g_ref[...] + _dot(ev, wc[...]) + b0[...], 0.0)
    o_ref[...] = _mlp_tail(h, w1[...], b1[...], w2[...], b2[...], gam[...], bet[...]) + ev


def _prep_body(x_ref, wa, wb, p_ref, q_ref):
    xv = x_ref[...]
    p_ref[...] = _dot(xv, wa[...])
    q_ref[...] = _dot(xv, wb[...])


def _node_body(a0_ref, a1_ref, x_ref, w0a, w0b, b0, w1, b1, w2, b2, gam, bet,
               u0a, u0b, nx_ref, pp_ref, qq_ref):
    xv = x_ref[...]
    aggr = a0_ref[...] + a1_ref[...]
    h = jnp.maximum(_dot(aggr, w0a[...]) + _dot(xv, w0b[...]) + b0[...], 0.0)
    xu = _mlp_tail(h, w1[...], b1[...], w2[...], b2[...], gam[...], bet[...])
    nx_ref[...] = xu + xv
    pp_ref[...] = _dot(xu, u0a[...])
    qq_ref[...] = _dot(xu, u0b[...])


BE = 2000
BN = 1000

_w_spec = pl.BlockSpec((D, D), lambda i: (0, 0))
_b_spec = pl.BlockSpec((1, D), lambda i: (0, 0))


def _rows_spec(rows):
    return pl.BlockSpec((rows, D), lambda i: (i, 0))


def _edge_mlp(body, g, e, wc, b0, w1, b1, w2, b2, gam, bet):
    return pl.pallas_call(
        body,
        grid=(E // BE,),
        in_specs=[_rows_spec(BE), _rows_spec(BE),
                  _w_spec, _b_spec, _w_spec, _b_spec, _w_spec, _b_spec,
                  _b_spec, _b_spec],
        out_specs=_rows_spec(BE),
        out_shape=jax.ShapeDtypeStruct((E, D), jnp.float32),
    )(g, e, wc, b0, w1, b1, w2, b2, gam, bet)


def _prep(x, wa, wb):
    return pl.pallas_call(
        _prep_body,
        grid=(N // BN,),
        in_specs=[_rows_spec(BN), _w_spec, _w_spec],
        out_specs=[_rows_spec(BN), _rows_spec(BN)],
        out_shape=[jax.ShapeDtypeStruct((N, D), jnp.float32)] * 2,
    )(x, wa, wb)


def _node_mlp(a0, a1, x, w0a, w0b, b0, w1, b1, w2, b2, gam, bet, u0a, u0b):
    return pl.pallas_call(
        _node_body,
        grid=(N // BN,),
        in_specs=[_rows_spec(BN), _rows_spec(BN), _rows_spec(BN),
                  _w_spec, _w_spec, _b_spec, _w_spec, _b_spec, _w_spec, _b_spec,
                  _b_spec, _b_spec, _w_spec, _w_spec],
        out_specs=[_rows_spec(BN)] * 3,
        out_shape=[jax.ShapeDtypeStruct((N, D), jnp.float32)] * 3,
    )(a0, a1, x, w0a, w0b, b0, w1, b1, w2, b2, gam, bet, u0a, u0b)


def kernel(x, edge_index, edge_features,
           eW0, eb0, eW1, eb1, eW2, eb2, eg, ebt,
           nW0, nb0, nW1, nb1, nW2, nb2, ng, nbt,
           uW0, ub0, uW1, ub1, uW2, ub2, ug, ubt):
    src = edge_index[0].reshape(NW, NCH, CHUNK)
    tgt = edge_index[1].reshape(NW, NCH, CHUNK)
    zeros = jnp.zeros((N, D), jnp.float32)

    def body(s, carry):
        xc, ec = carry
        eW0s = eW0[s]
        eA, eB, eC = eW0s[:D], eW0s[D:2 * D], eW0s[2 * D:]
        p, q = _prep(xc, eA, eB)
        g1 = _sc_gather_sum(p, q, tgt, src)
        m = _edge_mlp(_edge_mlp_body, g1, ec, eC,
                      eb0[s].reshape(1, D), eW1[s], eb1[s].reshape(1, D),
                      eW2[s], eb2[s].reshape(1, D),
                      eg[s].reshape(1, D), ebt[s].reshape(1, D))
        parts = _sc_scatter_add(m, tgt, zeros)
        nW0s = nW0[s]
        uW0s = uW0[s]
        nx, pp, qq = _node_mlp(parts[0], parts[1], xc,
                               nW0s[:D], nW0s[D:],
                               nb0[s].reshape(1, D), nW1[s],
                               nb1[s].reshape(1, D), nW2[s],
                               nb2[s].reshape(1, D),
                               ng[s].reshape(1, D), nbt[s].reshape(1, D),
                               uW0s[:D], uW0s[D:2 * D])
        g2 = _sc_gather_sum(pp, qq, src, tgt)
        ne = _edge_mlp(_u_mlp_body, g2, ec, uW0s[2 * D:],
                       ub0[s].reshape(1, D), uW1[s], ub1[s].reshape(1, D),
                       uW2[s], ub2[s].reshape(1, D),
                       ug[s].reshape(1, D), ubt[s].reshape(1, D))
        return (nx, ne)

    return lax.fori_loop(0, 10, body, (x, edge_features))


# speedup vs baseline: 2.7635x; 2.7635x over previous
import functools

import jax
import jax.numpy as jnp
from jax import lax
from jax.experimental import pallas as pl
from jax.experimental.pallas import tpu as pltpu
from jax.experimental.pallas import tpu_sc as plsc

N = 10000
E = 320000
D = 128

NC = 2
NS = 16
NW = NC * NS
EPW = E // NW
CHUNK = 80
NCH = EPW // CHUNK
NPAD = 10240
RPT = NPAD // NS

def _sc_mesh():
    return plsc.VectorSubcoreMesh(core_axis_name="c", subcore_axis_name="s",
                                  num_cores=NC, num_subcores=NS)


@functools.cache
def _gather_kernel():
    @functools.partial(
        pl.kernel,
        out_type=jax.ShapeDtypeStruct((E, D), jnp.float32),
        mesh=_sc_mesh(),
        scratch_types=[
            pltpu.VMEM((NCH, CHUNK), jnp.int32),
            pltpu.VMEM((NCH, CHUNK), jnp.int32),
            pltpu.VMEM((CHUNK, D), jnp.float32),
            pltpu.SemaphoreType.DMA,
        ],
    )
    def k(pa_hbm, pb_hbm, ia_hbm, ib_hbm, out_hbm, ia_v, ib_v, rows_v, sem):
        wid = lax.axis_index("c") * NS + lax.axis_index("s")
        base = wid * EPW
        pltpu.sync_copy(ia_hbm.at[wid], ia_v)
        pltpu.sync_copy(ib_hbm.at[wid], ib_v)

        def step(j, carry):
            pltpu.async_copy(pa_hbm.at[ia_v.at[j]], rows_v, sem).wait()
            pltpu.async_copy(pb_hbm.at[ib_v.at[j]], rows_v, sem, add=True).wait()
            pltpu.sync_copy(rows_v, out_hbm.at[pl.ds(base + j * CHUNK, CHUNK)])
            return carry

        lax.fori_loop(0, NCH, step, 0)

    return k


def _sc_gather_sum(pa, pb, ia, ib):
    return _gather_kernel()(pa, pb, ia, ib)


@functools.cache
def _scatter_kernel():
    @functools.partial(
        pl.kernel,
        out_type=jax.ShapeDtypeStruct((NC, NPAD, D), jnp.float32),
        mesh=_sc_mesh(),
        scratch_types=[
            pltpu.VMEM((NCH, CHUNK), jnp.int32),
            pltpu.VMEM((CHUNK, D), jnp.float32),
            pltpu.VMEM_SHARED((NPAD, D), jnp.float32),
        ],
    )
    def k(m_hbm, idx_hbm, zeros_hbm, out_hbm, idx_v, rows_v, acc_sh):
        cid = lax.axis_index("c")
        sid = lax.axis_index("s")
        wid = cid * NS + sid
        base = wid * EPW
        pltpu.sync_copy(zeros_hbm.at[pl.ds(sid * RPT, RPT)],
                        acc_sh.at[pl.ds(sid * RPT, RPT)])
        pltpu.sync_copy(idx_hbm.at[wid], idx_v)
        plsc.subcore_barrier()

        def step(j, carry):
            pltpu.sync_copy(m_hbm.at[pl.ds(base + j * CHUNK, CHUNK)], rows_v)
            pltpu.sync_copy(rows_v, acc_sh.at[idx_v.at[j]], add=True)
            return carry

        lax.fori_loop(0, NCH, step, 0)
        plsc.subcore_barrier()
        pltpu.sync_copy(acc_sh.at[pl.ds(sid * RPT, RPT)],
                        out_hbm.at[cid, pl.ds(sid * RPT, RPT)])

    return k


def _sc_scatter_add(m, idx, zeros):
    return _scatter_kernel()(m, idx, zeros)


def _dot(a, b):
    return jnp.dot(a, b, preferred_element_type=jnp.float32)


def _mlp_tail(h, w1, b1, w2, b2, gam, bet):
    h = jnp.maximum(_dot(h, w1) + b1, 0.0)
    h = _dot(h, w2) + b2
    mu = jnp.mean(h, axis=-1, keepdims=True)
    var = jnp.mean((h - mu) ** 2, axis=-1, keepdims=True)
    h = (h - mu) * lax.rsqrt(var + 1e-5)
    return h * gam + bet


def _edge_mlp_body(g_ref, e_ref, wc, b0, w1, b1, w2, b2, gam, bet, o_ref):
    h = jnp.maximum(g_ref[...] + _dot(e_ref[...], wc[...]) + b0[...], 0.0)
    o_ref[...] = _mlp_tail(h, w1[...], b1[...], w2[...], b2[...], gam[...], bet[...])


def _u_mlp_body(g_ref, e_ref, wc, b0, w1, b1, w2, b2, gam, bet, o_ref):
    ev = e_ref[...]
    h = jnp.maximum(g_ref[...] + _dot(ev, wc[...]) + b0[...], 0.0)
    o_ref[...] = _mlp_tail(h, w1[...], b1[...], w2[...], b2[...], gam[...], bet[...]) + ev


def _prep_body(x_ref, wa, wb, p_ref, q_ref):
    xv = x_ref[...]
    p_ref[...] = _dot(xv, wa[...])
    q_ref[...] = _dot(xv, wb[...])


def _node_body(a0_ref, a1_ref, x_ref, w0a, w0b, b0, w1, b1, w2, b2, gam, bet,
               u0a, u0b, nx_ref, pp_ref, qq_ref):
    xv = x_ref[...]
    aggr = a0_ref[...] + a1_ref[...]
    h = jnp.maximum(_dot(aggr, w0a[...]) + _dot(xv, w0b[...]) + b0[...], 0.0)
    xu = _mlp_tail(h, w1[...], b1[...], w2[...], b2[...], gam[...], bet[...])
    nx_ref[...] = xu + xv
    pp_ref[...] = _dot(xu, u0a[...])
    qq_ref[...] = _dot(xu, u0b[...])


BE = 2000
BN = 1000

_w_spec = pl.BlockSpec((D, D), lambda i: (0, 0))
_b_spec = pl.BlockSpec((1, D), lambda i: (0, 0))


def _rows_spec(rows):
    return pl.BlockSpec((rows, D), lambda i: (i, 0))


def _edge_mlp(body, g, e, wc, b0, w1, b1, w2, b2, gam, bet):
    return pl.pallas_call(
        body,
        grid=(E // BE,),
        in_specs=[_rows_spec(BE), _rows_spec(BE),
                  _w_spec, _b_spec, _w_spec, _b_spec, _w_spec, _b_spec,
                  _b_spec, _b_spec],
        out_specs=_rows_spec(BE),
        out_shape=jax.ShapeDtypeStruct((E, D), jnp.float32),
    )(g, e, wc, b0, w1, b1, w2, b2, gam, bet)


def _prep(x, wa, wb):
    return pl.pallas_call(
        _prep_body,
        grid=(N // BN,),
        in_specs=[_rows_spec(BN), _w_spec, _w_spec],
        out_specs=[_rows_spec(BN), _rows_spec(BN)],
        out_shape=[jax.ShapeDtypeStruct((N, D), jnp.float32)] * 2,
    )(x, wa, wb)


def _node_mlp(a0, a1, x, w0a, w0b, b0, w1, b1, w2, b2, gam, bet, u0a, u0b):
    return pl.pallas_call(
        _node_body,
        grid=(N // BN,),
        in_specs=[_rows_spec(BN), _rows_spec(BN), _rows_spec(BN),
                  _w_spec, _w_spec, _b_spec, _w_spec, _b_spec, _w_spec, _b_spec,
                  _b_spec, _b_spec, _w_spec, _w_spec],
        out_specs=[_rows_spec(BN)] * 3,
        out_shape=[jax.ShapeDtypeStruct((N, D), jnp.float32)] * 3,
    )(a0, a1, x, w0a, w0b, b0, w1, b1, w2, b2, gam, bet, u0a, u0b)


def kernel(x, edge_index, edge_features,
           eW0, eb0, eW1, eb1, eW2, eb2, eg, ebt,
           nW0, nb0, nW1, nb1, nW2, nb2, ng, nbt,
           uW0, ub0, uW1, ub1, uW2, ub2, ug, ubt):
    src = edge_index[0].reshape(NW, NCH, CHUNK)
    tgt = edge_index[1].reshape(NW, NCH, CHUNK)
    zeros = jnp.zeros((NPAD, D), jnp.float32)

    def body(s, carry):
        xc, ec = carry
        eW0s = eW0[s]
        eA, eB, eC = eW0s[:D], eW0s[D:2 * D], eW0s[2 * D:]
        p, q = _prep(xc, eA, eB)
        g1 = _sc_gather_sum(p, q, tgt, src)
        m = _edge_mlp(_edge_mlp_body, g1, ec, eC,
                      eb0[s].reshape(1, D), eW1[s], eb1[s].reshape(1, D),
                      eW2[s], eb2[s].reshape(1, D),
                      eg[s].reshape(1, D), ebt[s].reshape(1, D))
        parts = _sc_scatter_add(m, tgt, zeros)
        nW0s = nW0[s]
        uW0s = uW0[s]
        nx, pp, qq = _node_mlp(parts[0], parts[1], xc,
                               nW0s[:D], nW0s[D:],
                               nb0[s].reshape(1, D), nW1[s],
                               nb1[s].reshape(1, D), nW2[s],
                               nb2[s].reshape(1, D),
                               ng[s].reshape(1, D), nbt[s].reshape(1, D),
                               uW0s[:D], uW0s[D:2 * D])
        g2 = _sc_gather_sum(pp, qq, src, tgt)
        ne = _edge_mlp(_u_mlp_body, g2, ec, uW0s[2 * D:],
                       ub0[s].reshape(1, D), uW1[s], ub1[s].reshape(1, D),
                       uW2[s], ub2[s].reshape(1, D),
                       ug[s].reshape(1, D), ubt[s].reshape(1, D))
        return (nx, ne)

    return lax.fori_loop(0, 10, body, (x, edge_features))


# speedup vs baseline: 3.4224x; 1.2384x over previous
import functools

import jax
import jax.numpy as jnp
from jax import lax
from jax.experimental import pallas as pl
from jax.experimental.pallas import tpu as pltpu
from jax.experimental.pallas import tpu_sc as plsc

N = 10000
E = 320000
D = 128

NC = 2
NS = 16
NW = NC * NS
EPW = E // NW
CHUNK = 40
NCH = EPW // CHUNK
NPAD = 10240
RPT = NPAD // NS

def _sc_mesh():
    return plsc.VectorSubcoreMesh(core_axis_name="c", subcore_axis_name="s",
                                  num_cores=NC, num_subcores=NS)


GRP = 5
GROUP = GRP * CHUNK
NG = NCH // GRP
SNG = NCH


@functools.cache
def _gather_kernel():
    @functools.partial(
        pl.kernel,
        out_type=jax.ShapeDtypeStruct((E, D), jnp.float32),
        mesh=_sc_mesh(),
        scratch_types=[
            pltpu.VMEM((NCH, CHUNK), jnp.int32),
            pltpu.VMEM((NCH, CHUNK), jnp.int32),
            pltpu.VMEM((2, GROUP, D), jnp.float32),
            pltpu.SemaphoreType.DMA,
            pltpu.SemaphoreType.DMA,
            pltpu.SemaphoreType.DMA,
        ],
    )
    def k(pa_hbm, pb_hbm, ia_hbm, ib_hbm, out_hbm, ia_v, ib_v, buf, sa, sb, so):
        wid = lax.axis_index("c") * NS + lax.axis_index("s")
        base = wid * EPW
        pltpu.sync_copy(ia_hbm.at[wid], ia_v)
        pltpu.sync_copy(ib_hbm.at[wid], ib_v)

        def mk_a(g, r, i):
            return pltpu.make_async_copy(
                pa_hbm.at[ia_v.at[g * GRP + i]],
                buf.at[r, pl.ds(i * CHUNK, CHUNK)], sa)

        def mk_b(g, r, i):
            return pltpu.make_async_copy(
                pb_hbm.at[ib_v.at[g * GRP + i]],
                buf.at[r, pl.ds(i * CHUNK, CHUNK)], sb)

        def mk_o(g, r):
            return pltpu.make_async_copy(
                buf.at[r], out_hbm.at[pl.ds(base + g * GROUP, GROUP)], so)

        def issue_a(g, r):
            for i in range(GRP):
                mk_a(g, r, i).start()

        issue_a(0, 0)

        def body(g, carry):
            r = g % 2
            for i in range(GRP):
                mk_a(g, r, i).wait()
            for i in range(GRP):
                mk_b(g, r, i).start(add=True)

            @pl.when(g + 1 < NG)
            def _():
                @pl.when(g >= 1)
                def _():
                    mk_o(g - 1, 1 - r).wait()
                issue_a(g + 1, 1 - r)

            for i in range(GRP):
                mk_b(g, r, i).wait()
            mk_o(g, r).start()

            return carry

        lax.fori_loop(0, NG, body, 0)
        mk_o(NG - 2, NG % 2).wait()
        mk_o(NG - 1, (NG - 1) % 2).wait()

    return k


def _sc_gather_sum(pa, pb, ia, ib):
    return _gather_kernel()(pa, pb, ia, ib)


@functools.cache
def _scatter_kernel():
    @functools.partial(
        pl.kernel,
        out_type=jax.ShapeDtypeStruct((NC, NPAD, D), jnp.float32),
        mesh=_sc_mesh(),
        scratch_types=[
            pltpu.VMEM((2, CHUNK), jnp.int32),
            pltpu.VMEM((2, CHUNK, D), jnp.float32),
            pltpu.VMEM_SHARED((NPAD, D), jnp.float32),
            pltpu.SemaphoreType.DMA,
            pltpu.SemaphoreType.DMA,
            pltpu.SemaphoreType.DMA,
        ],
    )
    def k(m_hbm, idx_hbm, zeros_hbm, out_hbm, idx_v, buf, acc_sh, si, sx, ss):
        cid = lax.axis_index("c")
        sid = lax.axis_index("s")
        wid = cid * NS + sid
        base = wid * EPW
        pltpu.sync_copy(zeros_hbm.at[pl.ds(sid * RPT, RPT)],
                        acc_sh.at[pl.ds(sid * RPT, RPT)])
        plsc.subcore_barrier()

        def mk_i(g, r):
            return pltpu.make_async_copy(
                m_hbm.at[pl.ds(base + g * CHUNK, CHUNK)], buf.at[r], si)

        def mk_x(g, r):
            return pltpu.make_async_copy(idx_hbm.at[wid, g], idx_v.at[r], sx)

        def mk_s(g, r):
            return pltpu.make_async_copy(buf.at[r], acc_sh.at[idx_v.at[r]], ss)

        mk_i(0, 0).start()
        mk_x(0, 0).start()

        def body(g, carry):
            r = g % 2
            mk_i(g, r).wait()
            mk_x(g, r).wait()
            mk_s(g, r).start(add=True)

            @pl.when(g + 1 < SNG)
            def _():
                @pl.when(g >= 1)
                def _():
                    mk_s(g - 1, 1 - r).wait()
                mk_i(g + 1, 1 - r).start()
                mk_x(g + 1, 1 - r).start()

            return carry

        lax.fori_loop(0, SNG, body, 0)
        mk_s(SNG - 2, SNG % 2).wait()
        mk_s(SNG - 1, (SNG - 1) % 2).wait()
        plsc.subcore_barrier()
        pltpu.sync_copy(acc_sh.at[pl.ds(sid * RPT, RPT)],
                        out_hbm.at[cid, pl.ds(sid * RPT, RPT)])

    return k


def _sc_scatter_add(m, idx, zeros):
    return _scatter_kernel()(m, idx, zeros)


def _dot(a, b):
    return jnp.dot(a, b, preferred_element_type=jnp.float32)


def _mlp_tail(h, w1, b1, w2, b2, gam, bet):
    h = jnp.maximum(_dot(h, w1) + b1, 0.0)
    h = _dot(h, w2) + b2
    mu = jnp.mean(h, axis=-1, keepdims=True)
    var = jnp.mean((h - mu) ** 2, axis=-1, keepdims=True)
    h = (h - mu) * lax.rsqrt(var + 1e-5)
    return h * gam + bet


def _edge_mlp_body(g_ref, e_ref, wc, b0, w1, b1, w2, b2, gam, bet, o_ref):
    h = jnp.maximum(g_ref[...] + _dot(e_ref[...], wc[...]) + b0[...], 0.0)
    o_ref[...] = _mlp_tail(h, w1[...], b1[...], w2[...], b2[...], gam[...], bet[...])


def _u_mlp_body(g_ref, e_ref, wc, b0, w1, b1, w2, b2, gam, bet, o_ref):
    ev = e_ref[...]
    h = jnp.maximum(g_ref[...] + _dot(ev, wc[...]) + b0[...], 0.0)
    o_ref[...] = _mlp_tail(h, w1[...], b1[...], w2[...], b2[...], gam[...], bet[...]) + ev


def _prep_body(x_ref, wa, wb, p_ref, q_ref):
    xv = x_ref[...]
    p_ref[...] = _dot(xv, wa[...])
    q_ref[...] = _dot(xv, wb[...])


def _node_body(a0_ref, a1_ref, x_ref, w0a, w0b, b0, w1, b1, w2, b2, gam, bet,
               u0a, u0b, nx_ref, pp_ref, qq_ref):
    xv = x_ref[...]
    aggr = a0_ref[...] + a1_ref[...]
    h = jnp.maximum(_dot(aggr, w0a[...]) + _dot(xv, w0b[...]) + b0[...], 0.0)
    xu = _mlp_tail(h, w1[...], b1[...], w2[...], b2[...], gam[...], bet[...])
    nx_ref[...] = xu + xv
    pp_ref[...] = _dot(xu, u0a[...])
    qq_ref[...] = _dot(xu, u0b[...])


BE = 2000
BN = 1000

_w_spec = pl.BlockSpec((D, D), lambda i: (0, 0))
_b_spec = pl.BlockSpec((1, D), lambda i: (0, 0))


def _rows_spec(rows):
    return pl.BlockSpec((rows, D), lambda i: (i, 0))


def _edge_mlp(body, g, e, wc, b0, w1, b1, w2, b2, gam, bet):
    return pl.pallas_call(
        body,
        grid=(E // BE,),
        in_specs=[_rows_spec(BE), _rows_spec(BE),
                  _w_spec, _b_spec, _w_spec, _b_spec, _w_spec, _b_spec,
                  _b_spec, _b_spec],
        out_specs=_rows_spec(BE),
        out_shape=jax.ShapeDtypeStruct((E, D), jnp.float32),
    )(g, e, wc, b0, w1, b1, w2, b2, gam, bet)


def _prep(x, wa, wb):
    return pl.pallas_call(
        _prep_body,
        grid=(N // BN,),
        in_specs=[_rows_spec(BN), _w_spec, _w_spec],
        out_specs=[_rows_spec(BN), _rows_spec(BN)],
        out_shape=[jax.ShapeDtypeStruct((N, D), jnp.float32)] * 2,
    )(x, wa, wb)


def _node_mlp(a0, a1, x, w0a, w0b, b0, w1, b1, w2, b2, gam, bet, u0a, u0b):
    return pl.pallas_call(
        _node_body,
        grid=(N // BN,),
        in_specs=[_rows_spec(BN), _rows_spec(BN), _rows_spec(BN),
                  _w_spec, _w_spec, _b_spec, _w_spec, _b_spec, _w_spec, _b_spec,
                  _b_spec, _b_spec, _w_spec, _w_spec],
        out_specs=[_rows_spec(BN)] * 3,
        out_shape=[jax.ShapeDtypeStruct((N, D), jnp.float32)] * 3,
    )(a0, a1, x, w0a, w0b, b0, w1, b1, w2, b2, gam, bet, u0a, u0b)


def kernel(x, edge_index, edge_features,
           eW0, eb0, eW1, eb1, eW2, eb2, eg, ebt,
           nW0, nb0, nW1, nb1, nW2, nb2, ng, nbt,
           uW0, ub0, uW1, ub1, uW2, ub2, ug, ubt):
    src = edge_index[0].reshape(NW, NCH, CHUNK)
    tgt = edge_index[1].reshape(NW, NCH, CHUNK)
    zeros = jnp.zeros((NPAD, D), jnp.float32)

    def body(s, carry):
        xc, ec = carry
        eW0s = eW0[s]
        eA, eB, eC = eW0s[:D], eW0s[D:2 * D], eW0s[2 * D:]
        p, q = _prep(xc, eA, eB)
        g1 = _sc_gather_sum(p, q, tgt, src)
        m = _edge_mlp(_edge_mlp_body, g1, ec, eC,
                      eb0[s].reshape(1, D), eW1[s], eb1[s].reshape(1, D),
                      eW2[s], eb2[s].reshape(1, D),
                      eg[s].reshape(1, D), ebt[s].reshape(1, D))
        parts = _sc_scatter_add(m, tgt, zeros)
        nW0s = nW0[s]
        uW0s = uW0[s]
        nx, pp, qq = _node_mlp(parts[0], parts[1], xc,
                               nW0s[:D], nW0s[D:],
                               nb0[s].reshape(1, D), nW1[s],
                               nb1[s].reshape(1, D), nW2[s],
                               nb2[s].reshape(1, D),
                               ng[s].reshape(1, D), nbt[s].reshape(1, D),
                               uW0s[:D], uW0s[D:2 * D])
        g2 = _sc_gather_sum(pp, qq, src, tgt)
        ne = _edge_mlp(_u_mlp_body, g2, ec, uW0s[2 * D:],
                       ub0[s].reshape(1, D), uW1[s], ub1[s].reshape(1, D),
                       uW2[s], ub2[s].reshape(1, D),
                       ug[s].reshape(1, D), ubt[s].reshape(1, D))
        return (nx, ne)

    return lax.fori_loop(0, 10, body, (x, edge_features))


# speedup vs baseline: 4.0732x; 1.1902x over previous
import functools

import jax
import jax.numpy as jnp
from jax import lax
from jax.experimental import pallas as pl
from jax.experimental.pallas import tpu as pltpu
from jax.experimental.pallas import tpu_sc as plsc

N = 10000
E = 320000
D = 128

NC = 2
NS = 16
NW = NC * NS
EPW = E // NW
CHUNK = 40
NCH = EPW // CHUNK
NPAD = 10240
RPT = NPAD // NS

def _sc_mesh():
    return plsc.VectorSubcoreMesh(core_axis_name="c", subcore_axis_name="s",
                                  num_cores=NC, num_subcores=NS)


GRP = 5
GROUP = GRP * CHUNK
NG = NCH // GRP
SNG = NCH


@functools.cache
def _gather_kernel():
    @functools.partial(
        pl.kernel,
        out_type=jax.ShapeDtypeStruct((E, D), jnp.float32),
        mesh=_sc_mesh(),
        scratch_types=[
            pltpu.VMEM((NCH, CHUNK), jnp.int32),
            pltpu.VMEM((NCH, CHUNK), jnp.int32),
            pltpu.VMEM((2, GROUP, D), jnp.float32),
            pltpu.SemaphoreType.DMA,
            pltpu.SemaphoreType.DMA,
            pltpu.SemaphoreType.DMA,
        ],
    )
    def k(pa_hbm, pb_hbm, ia_hbm, ib_hbm, out_hbm, ia_v, ib_v, buf, sa, sb, so):
        wid = lax.axis_index("c") * NS + lax.axis_index("s")
        base = wid * EPW
        pltpu.sync_copy(ia_hbm.at[wid], ia_v)
        pltpu.sync_copy(ib_hbm.at[wid], ib_v)

        def mk_a(g, r, i):
            return pltpu.make_async_copy(
                pa_hbm.at[ia_v.at[g * GRP + i]],
                buf.at[r, pl.ds(i * CHUNK, CHUNK)], sa)

        def mk_b(g, r, i):
            return pltpu.make_async_copy(
                pb_hbm.at[ib_v.at[g * GRP + i]],
                buf.at[r, pl.ds(i * CHUNK, CHUNK)], sb)

        def mk_o(g, r):
            return pltpu.make_async_copy(
                buf.at[r], out_hbm.at[pl.ds(base + g * GROUP, GROUP)], so)

        def issue_a(g, r):
            for i in range(GRP):
                mk_a(g, r, i).start()

        issue_a(0, 0)

        def body(g, carry):
            r = g % 2
            for i in range(GRP):
                mk_a(g, r, i).wait()
            for i in range(GRP):
                mk_b(g, r, i).start(add=True)

            @pl.when(g + 1 < NG)
            def _():
                @pl.when(g >= 1)
                def _():
                    mk_o(g - 1, 1 - r).wait()
                issue_a(g + 1, 1 - r)

            for i in range(GRP):
                mk_b(g, r, i).wait()
            mk_o(g, r).start()

            return carry

        lax.fori_loop(0, NG, body, 0)
        mk_o(NG - 2, NG % 2).wait()
        mk_o(NG - 1, (NG - 1) % 2).wait()

    return k


def _sc_gather_sum(pa, pb, ia, ib):
    return _gather_kernel()(pa, pb, ia, ib)


@functools.cache
def _scatter_kernel():
    @functools.partial(
        pl.kernel,
        out_type=jax.ShapeDtypeStruct((NC, NPAD, D), jnp.float32),
        mesh=_sc_mesh(),
        scratch_types=[
            pltpu.VMEM((2, CHUNK), jnp.int32),
            pltpu.VMEM((2, CHUNK, D), jnp.float32),
            pltpu.VMEM_SHARED((NPAD, D), jnp.float32),
            pltpu.SemaphoreType.DMA,
            pltpu.SemaphoreType.DMA,
            pltpu.SemaphoreType.DMA,
        ],
    )
    def k(m_hbm, idx_hbm, zeros_hbm, out_hbm, idx_v, buf, acc_sh, si, sx, ss):
        cid = lax.axis_index("c")
        sid = lax.axis_index("s")
        wid = cid * NS + sid
        base = wid * EPW
        pltpu.sync_copy(zeros_hbm.at[pl.ds(sid * RPT, RPT)],
                        acc_sh.at[pl.ds(sid * RPT, RPT)])
        plsc.subcore_barrier()

        def mk_i(g, r):
            return pltpu.make_async_copy(
                m_hbm.at[pl.ds(base + g * CHUNK, CHUNK)], buf.at[r], si)

        def mk_x(g, r):
            return pltpu.make_async_copy(idx_hbm.at[wid, g], idx_v.at[r], sx)

        def mk_s(g, r):
            return pltpu.make_async_copy(buf.at[r], acc_sh.at[idx_v.at[r]], ss)

        mk_i(0, 0).start()
        mk_x(0, 0).start()

        def body(g, carry):
            r = g % 2
            mk_i(g, r).wait()
            mk_x(g, r).wait()
            mk_s(g, r).start(add=True)

            @pl.when(g + 1 < SNG)
            def _():
                @pl.when(g >= 1)
                def _():
                    mk_s(g - 1, 1 - r).wait()
                mk_i(g + 1, 1 - r).start()
                mk_x(g + 1, 1 - r).start()

            return carry

        lax.fori_loop(0, SNG, body, 0)
        mk_s(SNG - 2, SNG % 2).wait()
        mk_s(SNG - 1, (SNG - 1) % 2).wait()
        plsc.subcore_barrier()
        pltpu.sync_copy(acc_sh.at[pl.ds(sid * RPT, RPT)],
                        out_hbm.at[cid, pl.ds(sid * RPT, RPT)])

    return k


def _sc_scatter_add(m, idx, zeros):
    return _scatter_kernel()(m, idx, zeros)


def _dot(a, b):
    return jnp.dot(a, b, preferred_element_type=jnp.float32)


def _mlp_tail(h, w1, b1, w2, b2, gam, bet):
    h = jnp.maximum(_dot(h, w1) + b1, 0.0)
    h = _dot(h, w2) + b2
    mu = jnp.mean(h, axis=-1, keepdims=True)
    var = jnp.mean((h - mu) ** 2, axis=-1, keepdims=True)
    h = (h - mu) * lax.rsqrt(var + 1e-5)
    return h * gam + bet


def _edge_mlp_body(g_ref, e_ref, wc, b0, w1, b1, w2, b2, gam, bet, o_ref):
    h = jnp.maximum(g_ref[...] + _dot(e_ref[...], wc[...]) + b0[...], 0.0)
    o_ref[...] = _mlp_tail(h, w1[...], b1[...], w2[...], b2[...], gam[...], bet[...])


def _u_mlp_body(g_ref, e_ref, wc, b0, w1, b1, w2, b2, gam, bet, o_ref):
    ev = e_ref[...]
    h = jnp.maximum(g_ref[...] + _dot(ev, wc[...]) + b0[...], 0.0)
    o_ref[...] = _mlp_tail(h, w1[...], b1[...], w2[...], b2[...], gam[...], bet[...]) + ev


def _prep_body(x_ref, wa, wb, p_ref, q_ref):
    xv = x_ref[...]
    p_ref[...] = _dot(xv, wa[...])
    q_ref[...] = _dot(xv, wb[...])


def _node_body(a0_ref, a1_ref, x_ref, w0a, w0b, b0, w1, b1, w2, b2, gam, bet,
               u0a, u0b, ea, eb, nx_ref, pp_ref, qq_ref, pn_ref, qn_ref):
    xv = x_ref[...]
    aggr = a0_ref[...] + a1_ref[...]
    h = jnp.maximum(_dot(aggr, w0a[...]) + _dot(xv, w0b[...]) + b0[...], 0.0)
    xu = _mlp_tail(h, w1[...], b1[...], w2[...], b2[...], gam[...], bet[...])
    nx = xu + xv
    nx_ref[...] = nx
    pp_ref[...] = _dot(xu, u0a[...])
    qq_ref[...] = _dot(xu, u0b[...])
    pn_ref[...] = _dot(nx, ea[...])
    qn_ref[...] = _dot(nx, eb[...])


BE = 2000
BN = 1000

_w_spec = pl.BlockSpec((D, D), lambda i: (0, 0))
_b_spec = pl.BlockSpec((1, D), lambda i: (0, 0))


def _rows_spec(rows):
    return pl.BlockSpec((rows, D), lambda i: (i, 0))


def _edge_mlp(body, g, e, wc, b0, w1, b1, w2, b2, gam, bet):
    return pl.pallas_call(
        body,
        grid=(E // BE,),
        in_specs=[_rows_spec(BE), _rows_spec(BE),
                  _w_spec, _b_spec, _w_spec, _b_spec, _w_spec, _b_spec,
                  _b_spec, _b_spec],
        out_specs=_rows_spec(BE),
        out_shape=jax.ShapeDtypeStruct((E, D), jnp.float32),
    )(g, e, wc, b0, w1, b1, w2, b2, gam, bet)


def _prep(x, wa, wb):
    return pl.pallas_call(
        _prep_body,
        grid=(N // BN,),
        in_specs=[_rows_spec(BN), _w_spec, _w_spec],
        out_specs=[_rows_spec(BN), _rows_spec(BN)],
        out_shape=[jax.ShapeDtypeStruct((N, D), jnp.float32)] * 2,
    )(x, wa, wb)


def _node_mlp(a0, a1, x, w0a, w0b, b0, w1, b1, w2, b2, gam, bet, u0a, u0b, ea, eb):
    return pl.pallas_call(
        _node_body,
        grid=(N // BN,),
        in_specs=[_rows_spec(BN), _rows_spec(BN), _rows_spec(BN),
                  _w_spec, _w_spec, _b_spec, _w_spec, _b_spec, _w_spec, _b_spec,
                  _b_spec, _b_spec, _w_spec, _w_spec, _w_spec, _w_spec],
        out_specs=[_rows_spec(BN)] * 5,
        out_shape=[jax.ShapeDtypeStruct((N, D), jnp.float32)] * 5,
    )(a0, a1, x, w0a, w0b, b0, w1, b1, w2, b2, gam, bet, u0a, u0b, ea, eb)


def kernel(x, edge_index, edge_features,
           eW0, eb0, eW1, eb1, eW2, eb2, eg, ebt,
           nW0, nb0, nW1, nb1, nW2, nb2, ng, nbt,
           uW0, ub0, uW1, ub1, uW2, ub2, ug, ubt):
    src = edge_index[0].reshape(NW, NCH, CHUNK)
    tgt = edge_index[1].reshape(NW, NCH, CHUNK)
    zeros = jnp.zeros((NPAD, D), jnp.float32)

    def b(v):
        return v.reshape(1, D)

    xc, ec = x, edge_features
    p, q = _prep(xc, eW0[0][:D], eW0[0][D:2 * D])
    g1 = _sc_gather_sum(p, q, tgt, src)
    for s in range(10):
        eW0s, nW0s, uW0s = eW0[s], nW0[s], uW0[s]
        eW0n = eW0[s + 1] if s < 9 else eW0s
        m = _edge_mlp(_edge_mlp_body, g1, ec, eW0s[2 * D:],
                      b(eb0[s]), eW1[s], b(eb1[s]), eW2[s], b(eb2[s]),
                      b(eg[s]), b(ebt[s]))
        parts = _sc_scatter_add(m, tgt, zeros)
        nx, pp, qq, pn, qn = _node_mlp(
            parts[0], parts[1], xc,
            nW0s[:D], nW0s[D:], b(nb0[s]), nW1[s], b(nb1[s]), nW2[s],
            b(nb2[s]), b(ng[s]), b(nbt[s]),
            uW0s[:D], uW0s[D:2 * D], eW0n[:D], eW0n[D:2 * D])
        g2 = _sc_gather_sum(pp, qq, src, tgt)
        if s < 9:
            g1 = _sc_gather_sum(pn, qn, tgt, src)
        ne = _edge_mlp(_u_mlp_body, g2, ec, uW0s[2 * D:],
                       b(ub0[s]), uW1[s], b(ub1[s]), uW2[s], b(ub2[s]),
                       b(ug[s]), b(ubt[s]))
        xc, ec = nx, ne
    return (xc, ec)


# speedup vs baseline: 4.8180x; 1.1828x over previous
import functools

import jax
import jax.numpy as jnp
from jax import lax
from jax.experimental import pallas as pl
from jax.experimental.pallas import tpu as pltpu
from jax.experimental.pallas import tpu_sc as plsc

N = 10000
E = 320000
D = 128

NC = 2
NS = 16
NW = NC * NS
EPW = E // NW
CHUNK = 40
NCH = EPW // CHUNK
NPAD = 10240
RPT = NPAD // NS

def _sc_mesh():
    return plsc.VectorSubcoreMesh(core_axis_name="c", subcore_axis_name="s",
                                  num_cores=NC, num_subcores=NS)


GRP = 5
GROUP = GRP * CHUNK
EH = E // 2


@functools.cache
def _gather_kernel(ne):
    epw = ne // NW
    nch = epw // CHUNK
    ng = nch // GRP

    @functools.partial(
        pl.kernel,
        out_type=jax.ShapeDtypeStruct((ne, D), jnp.float32),
        mesh=_sc_mesh(),
        scratch_types=[
            pltpu.VMEM((nch, CHUNK), jnp.int32),
            pltpu.VMEM((nch, CHUNK), jnp.int32),
            pltpu.VMEM((2, GROUP, D), jnp.float32),
            pltpu.SemaphoreType.DMA,
            pltpu.SemaphoreType.DMA,
            pltpu.SemaphoreType.DMA,
        ],
    )
    def k(pa_hbm, pb_hbm, ia_hbm, ib_hbm, out_hbm, ia_v, ib_v, buf, sa, sb, so):
        wid = lax.axis_index("c") * NS + lax.axis_index("s")
        base = wid * epw
        pltpu.sync_copy(ia_hbm.at[wid], ia_v)
        pltpu.sync_copy(ib_hbm.at[wid], ib_v)

        def mk_a(g, r, i):
            return pltpu.make_async_copy(
                pa_hbm.at[ia_v.at[g * GRP + i]],
                buf.at[r, pl.ds(i * CHUNK, CHUNK)], sa)

        def mk_b(g, r, i):
            return pltpu.make_async_copy(
                pb_hbm.at[ib_v.at[g * GRP + i]],
                buf.at[r, pl.ds(i * CHUNK, CHUNK)], sb)

        def mk_o(g, r):
            return pltpu.make_async_copy(
                buf.at[r], out_hbm.at[pl.ds(base + g * GROUP, GROUP)], so)

        def issue_a(g, r):
            for i in range(GRP):
                mk_a(g, r, i).start()

        issue_a(0, 0)

        def body(g, carry):
            r = g % 2
            for i in range(GRP):
                mk_a(g, r, i).wait()
            for i in range(GRP):
                mk_b(g, r, i).start(add=True)

            @pl.when(g + 1 < ng)
            def _():
                @pl.when(g >= 1)
                def _():
                    mk_o(g - 1, 1 - r).wait()
                issue_a(g + 1, 1 - r)

            for i in range(GRP):
                mk_b(g, r, i).wait()
            mk_o(g, r).start()

            return carry

        lax.fori_loop(0, ng, body, 0)
        mk_o(ng - 2, ng % 2).wait()
        mk_o(ng - 1, (ng - 1) % 2).wait()

    return k


def _sc_gather_sum(pa, pb, ia, ib):
    return _gather_kernel(ia.shape[0] * ia.shape[1] * ia.shape[2])(pa, pb, ia, ib)


@functools.cache
def _scatter_kernel(ne):
    epw = ne // NW
    sng = epw // CHUNK

    @functools.partial(
        pl.kernel,
        out_type=jax.ShapeDtypeStruct((NC, NPAD, D), jnp.float32),
        mesh=_sc_mesh(),
        scratch_types=[
            pltpu.VMEM((3, CHUNK), jnp.int32),
            pltpu.VMEM((3, CHUNK, D), jnp.float32),
            pltpu.VMEM_SHARED((NPAD, D), jnp.float32),
            pltpu.SemaphoreType.DMA,
            pltpu.SemaphoreType.DMA,
            pltpu.SemaphoreType.DMA,
        ],
    )
    def k(m_hbm, idx_hbm, zeros_hbm, out_hbm, idx_v, buf, acc_sh, si, sx, ss):
        cid = lax.axis_index("c")
        sid = lax.axis_index("s")
        wid = cid * NS + sid
        base = wid * epw
        pltpu.sync_copy(zeros_hbm.at[pl.ds(sid * RPT, RPT)],
                        acc_sh.at[pl.ds(sid * RPT, RPT)])
        plsc.subcore_barrier()

        def mk_i(g, r):
            return pltpu.make_async_copy(
                m_hbm.at[pl.ds(base + g * CHUNK, CHUNK)], buf.at[r], si)

        def mk_x(g, r):
            return pltpu.make_async_copy(idx_hbm.at[wid, g], idx_v.at[r], sx)

        def mk_s(g, r):
            return pltpu.make_async_copy(buf.at[r], acc_sh.at[idx_v.at[r]], ss)

        for gg in range(2):
            mk_i(gg, gg).start()
            mk_x(gg, gg).start()

        def body(g, carry):
            r = g % 3
            mk_i(g, r).wait()
            mk_x(g, r).wait()
            mk_s(g, r).start(add=True)

            @pl.when(g + 2 < sng)
            def _():
                @pl.when(g >= 1)
                def _():
                    mk_s(g - 1, (g - 1) % 3).wait()
                mk_i(g + 2, (g + 2) % 3).start()
                mk_x(g + 2, (g + 2) % 3).start()

            return carry

        lax.fori_loop(0, sng, body, 0)
        mk_s(sng - 3, (sng - 3) % 3).wait()
        mk_s(sng - 2, (sng - 2) % 3).wait()
        mk_s(sng - 1, (sng - 1) % 3).wait()
        plsc.subcore_barrier()
        pltpu.sync_copy(acc_sh.at[pl.ds(sid * RPT, RPT)],
                        out_hbm.at[cid, pl.ds(sid * RPT, RPT)])

    return k


def _sc_scatter_add(m, idx, zeros):
    return _scatter_kernel(m.shape[0])(m, idx, zeros)


def _dot(a, b):
    return jnp.dot(a, b, preferred_element_type=jnp.float32)


def _mlp_tail(h, w1, b1, w2, b2, gam, bet):
    h = jnp.maximum(_dot(h, w1) + b1, 0.0)
    h = _dot(h, w2) + b2
    mu = jnp.mean(h, axis=-1, keepdims=True)
    var = jnp.mean((h - mu) ** 2, axis=-1, keepdims=True)
    h = (h - mu) * lax.rsqrt(var + 1e-5)
    return h * gam + bet


def _edge_mlp_body(g_ref, e_ref, wc, b0, w1, b1, w2, b2, gam, bet, o_ref):
    h = jnp.maximum(g_ref[...] + _dot(e_ref[...], wc[...]) + b0[...], 0.0)
    o_ref[...] = _mlp_tail(h, w1[...], b1[...], w2[...], b2[...], gam[...], bet[...])


def _u_mlp_body(g_ref, e_ref, wc, b0, w1, b1, w2, b2, gam, bet, o_ref):
    ev = e_ref[...]
    h = jnp.maximum(g_ref[...] + _dot(ev, wc[...]) + b0[...], 0.0)
    o_ref[...] = _mlp_tail(h, w1[...], b1[...], w2[...], b2[...], gam[...], bet[...]) + ev


def _prep_body(x_ref, wa, wb, p_ref, q_ref):
    xv = x_ref[...]
    p_ref[...] = _dot(xv, wa[...])
    q_ref[...] = _dot(xv, wb[...])


def _node_body(a0_ref, a1_ref, a2_ref, a3_ref, x_ref, w0a, w0b, b0, w1, b1,
               w2, b2, gam, bet, u0a, u0b, ea, eb,
               nx_ref, pp_ref, qq_ref, pn_ref, qn_ref):
    xv = x_ref[...]
    aggr = (a0_ref[...] + a1_ref[...]) + (a2_ref[...] + a3_ref[...])
    h = jnp.maximum(_dot(aggr, w0a[...]) + _dot(xv, w0b[...]) + b0[...], 0.0)
    xu = _mlp_tail(h, w1[...], b1[...], w2[...], b2[...], gam[...], bet[...])
    nx = xu + xv
    nx_ref[...] = nx
    pp_ref[...] = _dot(xu, u0a[...])
    qq_ref[...] = _dot(xu, u0b[...])
    pn_ref[...] = _dot(nx, ea[...])
    qn_ref[...] = _dot(nx, eb[...])


BE = 2000
BN = 1000

_w_spec = pl.BlockSpec((D, D), lambda i: (0, 0))
_b_spec = pl.BlockSpec((1, D), lambda i: (0, 0))


def _rows_spec(rows):
    return pl.BlockSpec((rows, D), lambda i: (i, 0))


def _edge_mlp(body, g, e, wc, b0, w1, b1, w2, b2, gam, bet):
    ne = g.shape[0]
    return pl.pallas_call(
        body,
        grid=(ne // BE,),
        in_specs=[_rows_spec(BE), _rows_spec(BE),
                  _w_spec, _b_spec, _w_spec, _b_spec, _w_spec, _b_spec,
                  _b_spec, _b_spec],
        out_specs=_rows_spec(BE),
        out_shape=jax.ShapeDtypeStruct((ne, D), jnp.float32),
    )(g, e, wc, b0, w1, b1, w2, b2, gam, bet)


def _prep(x, wa, wb):
    return pl.pallas_call(
        _prep_body,
        grid=(N // BN,),
        in_specs=[_rows_spec(BN), _w_spec, _w_spec],
        out_specs=[_rows_spec(BN), _rows_spec(BN)],
        out_shape=[jax.ShapeDtypeStruct((N, D), jnp.float32)] * 2,
    )(x, wa, wb)


def _node_mlp(a0, a1, a2, a3, x, w0a, w0b, b0, w1, b1, w2, b2, gam, bet,
              u0a, u0b, ea, eb):
    return pl.pallas_call(
        _node_body,
        grid=(N // BN,),
        in_specs=[_rows_spec(BN)] * 5
        + [_w_spec, _w_spec, _b_spec, _w_spec, _b_spec, _w_spec, _b_spec,
           _b_spec, _b_spec, _w_spec, _w_spec, _w_spec, _w_spec],
        out_specs=[_rows_spec(BN)] * 5,
        out_shape=[jax.ShapeDtypeStruct((N, D), jnp.float32)] * 5,
    )(a0, a1, a2, a3, x, w0a, w0b, b0, w1, b1, w2, b2, gam, bet, u0a, u0b, ea, eb)


def kernel(x, edge_index, edge_features,
           eW0, eb0, eW1, eb1, eW2, eb2, eg, ebt,
           nW0, nb0, nW1, nb1, nW2, nb2, ng, nbt,
           uW0, ub0, uW1, ub1, uW2, ub2, ug, ubt):
    hch = EH // (NW * CHUNK)
    src_h = [edge_index[0][h * EH:(h + 1) * EH].reshape(NW, hch, CHUNK)
             for h in range(2)]
    tgt_h = [edge_index[1][h * EH:(h + 1) * EH].reshape(NW, hch, CHUNK)
             for h in range(2)]
    e_h = [edge_features[:EH], edge_features[EH:]]
    zeros = jnp.zeros((NPAD, D), jnp.float32)

    def b(v):
        return v.reshape(1, D)

    xc = x
    p, q = _prep(xc, eW0[0][:D], eW0[0][D:2 * D])
    g1 = [_sc_gather_sum(p, q, tgt_h[0], src_h[0]),
          _sc_gather_sum(p, q, tgt_h[1], src_h[1])]
    for s in range(10):
        eW0s, nW0s, uW0s = eW0[s], nW0[s], uW0[s]
        eW0n = eW0[s + 1] if s < 9 else eW0s
        parts = []
        for h in range(2):
            m = _edge_mlp(_edge_mlp_body, g1[h], e_h[h], eW0s[2 * D:],
                          b(eb0[s]), eW1[s], b(eb1[s]), eW2[s], b(eb2[s]),
                          b(eg[s]), b(ebt[s]))
            parts.append(_sc_scatter_add(m, tgt_h[h], zeros))
        nx, pp, qq, pn, qn = _node_mlp(
            parts[0][0], parts[0][1], parts[1][0], parts[1][1], xc,
            nW0s[:D], nW0s[D:], b(nb0[s]), nW1[s], b(nb1[s]), nW2[s],
            b(nb2[s]), b(ng[s]), b(nbt[s]),
            uW0s[:D], uW0s[D:2 * D], eW0n[:D], eW0n[D:2 * D])
        ne_h = [None, None]
        g1n = [None, None]
        for h in range(2):
            g2 = _sc_gather_sum(pp, qq, src_h[h], tgt_h[h])
            if s < 9:
                g1n[h] = _sc_gather_sum(pn, qn, tgt_h[h], src_h[h])
            ne_h[h] = _edge_mlp(_u_mlp_body, g2, e_h[h], uW0s[2 * D:],
                                b(ub0[s]), uW1[s], b(ub1[s]), uW2[s],
                                b(ub2[s]), b(ug[s]), b(ubt[s]))
        xc, e_h, g1 = nx, ne_h, g1n
    return (xc, jnp.concatenate(e_h, axis=0))


# speedup vs baseline: 5.0942x; 1.0573x over previous
import functools

import jax
import jax.numpy as jnp
from jax import lax
from jax.experimental import pallas as pl
from jax.experimental.pallas import tpu as pltpu
from jax.experimental.pallas import tpu_sc as plsc

N = 10000
E = 320000
D = 128

NC = 2
NS = 16
NW = NC * NS
EPW = E // NW
CHUNK = 40
NCH = EPW // CHUNK
NPAD = 10240
RPT = NPAD // NS

def _sc_mesh():
    return plsc.VectorSubcoreMesh(core_axis_name="c", subcore_axis_name="s",
                                  num_cores=NC, num_subcores=NS)


GRP = 5
GROUP = GRP * CHUNK
EH = E // 2


@functools.cache
def _gather_kernel(ne):
    epw = ne // NW
    nch = epw // CHUNK
    ng = nch // GRP

    @functools.partial(
        pl.kernel,
        out_type=jax.ShapeDtypeStruct((ne, D), jnp.float32),
        mesh=_sc_mesh(),
        scratch_types=[
            pltpu.VMEM((nch, CHUNK), jnp.int32),
            pltpu.VMEM((nch, CHUNK), jnp.int32),
            pltpu.VMEM((2, GROUP, D), jnp.float32),
            pltpu.SemaphoreType.DMA,
            pltpu.SemaphoreType.DMA,
            pltpu.SemaphoreType.DMA,
        ],
    )
    def k(pa_hbm, pb_hbm, ia_hbm, ib_hbm, out_hbm, ia_v, ib_v, buf, sa, sb, so):
        wid = lax.axis_index("c") * NS + lax.axis_index("s")
        base = wid * epw
        pltpu.sync_copy(ia_hbm.at[wid], ia_v)
        pltpu.sync_copy(ib_hbm.at[wid], ib_v)

        def mk_a(g, r, i):
            return pltpu.make_async_copy(
                pa_hbm.at[ia_v.at[g * GRP + i]],
                buf.at[r, pl.ds(i * CHUNK, CHUNK)], sa)

        def mk_b(g, r, i):
            return pltpu.make_async_copy(
                pb_hbm.at[ib_v.at[g * GRP + i]],
                buf.at[r, pl.ds(i * CHUNK, CHUNK)], sb)

        def mk_o(g, r):
            return pltpu.make_async_copy(
                buf.at[r], out_hbm.at[pl.ds(base + g * GROUP, GROUP)], so)

        def issue_a(g, r):
            for i in range(GRP):
                mk_a(g, r, i).start()

        issue_a(0, 0)

        def body(g, carry):
            r = g % 2
            for i in range(GRP):
                mk_a(g, r, i).wait()
            for i in range(GRP):
                mk_b(g, r, i).start(add=True)

            @pl.when(g + 1 < ng)
            def _():
                @pl.when(g >= 1)
                def _():
                    mk_o(g - 1, 1 - r).wait()
                issue_a(g + 1, 1 - r)

            for i in range(GRP):
                mk_b(g, r, i).wait()
            mk_o(g, r).start()

            return carry

        lax.fori_loop(0, ng, body, 0)
        mk_o(ng - 2, ng % 2).wait()
        mk_o(ng - 1, (ng - 1) % 2).wait()

    return k


def _sc_gather_sum(pa, pb, ia, ib):
    return _gather_kernel(ia.shape[0] * ia.shape[1] * ia.shape[2])(pa, pb, ia, ib)


@functools.cache
def _scatter_kernel(ne):
    epw = ne // NW
    sng = epw // CHUNK

    @functools.partial(
        pl.kernel,
        out_type=jax.ShapeDtypeStruct((NC, NPAD, D), jnp.float32),
        mesh=_sc_mesh(),
        scratch_types=[
            pltpu.VMEM((3, CHUNK), jnp.int32),
            pltpu.VMEM((3, CHUNK, D), jnp.float32),
            pltpu.VMEM_SHARED((NPAD, D), jnp.float32),
            pltpu.SemaphoreType.DMA,
            pltpu.SemaphoreType.DMA,
            pltpu.SemaphoreType.DMA,
        ],
    )
    def k(m_hbm, idx_hbm, zeros_hbm, out_hbm, idx_v, buf, acc_sh, si, sx, ss):
        cid = lax.axis_index("c")
        sid = lax.axis_index("s")
        wid = cid * NS + sid
        base = wid * epw
        pltpu.sync_copy(zeros_hbm.at[pl.ds(sid * RPT, RPT)],
                        acc_sh.at[pl.ds(sid * RPT, RPT)])
        plsc.subcore_barrier()

        def mk_i(g, r):
            return pltpu.make_async_copy(
                m_hbm.at[pl.ds(base + g * CHUNK, CHUNK)], buf.at[r], si)

        def mk_x(g, r):
            return pltpu.make_async_copy(idx_hbm.at[wid, g], idx_v.at[r], sx)

        def mk_s(g, r):
            return pltpu.make_async_copy(buf.at[r], acc_sh.at[idx_v.at[r]], ss)

        for gg in range(2):
            mk_i(gg, gg).start()
            mk_x(gg, gg).start()

        def body(g, carry):
            r = g % 3
            mk_i(g, r).wait()
            mk_x(g, r).wait()
            mk_s(g, r).start(add=True)

            @pl.when(g + 2 < sng)
            def _():
                @pl.when(g >= 1)
                def _():
                    mk_s(g - 1, (g - 1) % 3).wait()
                mk_i(g + 2, (g + 2) % 3).start()
                mk_x(g + 2, (g + 2) % 3).start()

            return carry

        lax.fori_loop(0, sng, body, 0)
        mk_s(sng - 3, (sng - 3) % 3).wait()
        mk_s(sng - 2, (sng - 2) % 3).wait()
        mk_s(sng - 1, (sng - 1) % 3).wait()
        plsc.subcore_barrier()
        pltpu.sync_copy(acc_sh.at[pl.ds(sid * RPT, RPT)],
                        out_hbm.at[cid, pl.ds(sid * RPT, RPT)])

    return k


def _sc_scatter_add(m, idx, zeros):
    return _scatter_kernel(m.shape[0])(m, idx, zeros)


def _dot(a, b):
    return jnp.dot(a.astype(jnp.bfloat16), b.astype(jnp.bfloat16),
                   preferred_element_type=jnp.float32)


def _mlp_tail(h, w1, b1, w2, b2, gam, bet):
    h = jnp.maximum(_dot(h, w1) + b1, 0.0)
    h = _dot(h, w2) + b2
    mu = jnp.mean(h, axis=-1, keepdims=True)
    var = jnp.mean((h - mu) ** 2, axis=-1, keepdims=True)
    h = (h - mu) * lax.rsqrt(var + 1e-5)
    return h * gam + bet


def _edge_mlp_body(g_ref, e_ref, wc, b0, w1, b1, w2, b2, gam, bet, o_ref):
    h = jnp.maximum(g_ref[...] + _dot(e_ref[...], wc[...]) + b0[...], 0.0)
    o_ref[...] = _mlp_tail(h, w1[...], b1[...], w2[...], b2[...], gam[...], bet[...])


def _u_mlp_body(g_ref, e_ref, wc, b0, w1, b1, w2, b2, gam, bet, o_ref):
    ev = e_ref[...]
    h = jnp.maximum(g_ref[...] + _dot(ev, wc[...]) + b0[...], 0.0)
    o_ref[...] = _mlp_tail(h, w1[...], b1[...], w2[...], b2[...], gam[...], bet[...]) + ev


def _prep_body(x_ref, wa, wb, p_ref, q_ref):
    xv = x_ref[...]
    p_ref[...] = _dot(xv, wa[...])
    q_ref[...] = _dot(xv, wb[...])


def _node_body(a0_ref, a1_ref, a2_ref, a3_ref, x_ref, w0a, w0b, b0, w1, b1,
               w2, b2, gam, bet, u0a, u0b, ea, eb,
               nx_ref, pp_ref, qq_ref, pn_ref, qn_ref):
    xv = x_ref[...]
    aggr = (a0_ref[...] + a1_ref[...]) + (a2_ref[...] + a3_ref[...])
    h = jnp.maximum(_dot(aggr, w0a[...]) + _dot(xv, w0b[...]) + b0[...], 0.0)
    xu = _mlp_tail(h, w1[...], b1[...], w2[...], b2[...], gam[...], bet[...])
    nx = xu + xv
    nx_ref[...] = nx
    pp_ref[...] = _dot(xu, u0a[...])
    qq_ref[...] = _dot(xu, u0b[...])
    pn_ref[...] = _dot(nx, ea[...])
    qn_ref[...] = _dot(nx, eb[...])


BE = 4000
BN = 1000

_w_spec = pl.BlockSpec((D, D), lambda i: (0, 0))
_b_spec = pl.BlockSpec((1, D), lambda i: (0, 0))


def _rows_spec(rows):
    return pl.BlockSpec((rows, D), lambda i: (i, 0))


def _edge_mlp(body, g, e, wc, b0, w1, b1, w2, b2, gam, bet):
    ne = g.shape[0]
    return pl.pallas_call(
        body,
        grid=(ne // BE,),
        in_specs=[_rows_spec(BE), _rows_spec(BE),
                  _w_spec, _b_spec, _w_spec, _b_spec, _w_spec, _b_spec,
                  _b_spec, _b_spec],
        out_specs=_rows_spec(BE),
        out_shape=jax.ShapeDtypeStruct((ne, D), jnp.float32),
    )(g, e, wc, b0, w1, b1, w2, b2, gam, bet)


def _prep(x, wa, wb):
    return pl.pallas_call(
        _prep_body,
        grid=(N // BN,),
        in_specs=[_rows_spec(BN), _w_spec, _w_spec],
        out_specs=[_rows_spec(BN), _rows_spec(BN)],
        out_shape=[jax.ShapeDtypeStruct((N, D), jnp.float32)] * 2,
    )(x, wa, wb)


def _node_mlp(a0, a1, a2, a3, x, w0a, w0b, b0, w1, b1, w2, b2, gam, bet,
              u0a, u0b, ea, eb):
    return pl.pallas_call(
        _node_body,
        grid=(N // BN,),
        in_specs=[_rows_spec(BN)] * 5
        + [_w_spec, _w_spec, _b_spec, _w_spec, _b_spec, _w_spec, _b_spec,
           _b_spec, _b_spec, _w_spec, _w_spec, _w_spec, _w_spec],
        out_specs=[_rows_spec(BN)] * 5,
        out_shape=[jax.ShapeDtypeStruct((N, D), jnp.float32)] * 5,
    )(a0, a1, a2, a3, x, w0a, w0b, b0, w1, b1, w2, b2, gam, bet, u0a, u0b, ea, eb)


def kernel(x, edge_index, edge_features,
           eW0, eb0, eW1, eb1, eW2, eb2, eg, ebt,
           nW0, nb0, nW1, nb1, nW2, nb2, ng, nbt,
           uW0, ub0, uW1, ub1, uW2, ub2, ug, ubt):
    hch = EH // (NW * CHUNK)
    src_h = [edge_index[0][h * EH:(h + 1) * EH].reshape(NW, hch, CHUNK)
             for h in range(2)]
    tgt_h = [edge_index[1][h * EH:(h + 1) * EH].reshape(NW, hch, CHUNK)
             for h in range(2)]
    e_h = [edge_features[:EH], edge_features[EH:]]
    zeros = jnp.zeros((NPAD, D), jnp.float32)

    def b(v):
        return v.reshape(1, D)

    xc = x
    p, q = _prep(xc, eW0[0][:D], eW0[0][D:2 * D])
    g1 = [_sc_gather_sum(p, q, tgt_h[0], src_h[0]),
          _sc_gather_sum(p, q, tgt_h[1], src_h[1])]
    for s in range(10):
        eW0s, nW0s, uW0s = eW0[s], nW0[s], uW0[s]
        eW0n = eW0[s + 1] if s < 9 else eW0s
        parts = []
        for h in range(2):
            m = _edge_mlp(_edge_mlp_body, g1[h], e_h[h], eW0s[2 * D:],
                          b(eb0[s]), eW1[s], b(eb1[s]), eW2[s], b(eb2[s]),
                          b(eg[s]), b(ebt[s]))
            parts.append(_sc_scatter_add(m, tgt_h[h], zeros))
        nx, pp, qq, pn, qn = _node_mlp(
            parts[0][0], parts[0][1], parts[1][0], parts[1][1], xc,
            nW0s[:D], nW0s[D:], b(nb0[s]), nW1[s], b(nb1[s]), nW2[s],
            b(nb2[s]), b(ng[s]), b(nbt[s]),
            uW0s[:D], uW0s[D:2 * D], eW0n[:D], eW0n[D:2 * D])
        ne_h = [None, None]
        g1n = [None, None]
        for h in range(2):
            g2 = _sc_gather_sum(pp, qq, src_h[h], tgt_h[h])
            if s < 9:
                g1n[h] = _sc_gather_sum(pn, qn, tgt_h[h], src_h[h])
            ne_h[h] = _edge_mlp(_u_mlp_body, g2, e_h[h], uW0s[2 * D:],
                                b(ub0[s]), uW1[s], b(ub1[s]), uW2[s],
                                b(ub2[s]), b(ug[s]), b(ubt[s]))
        xc, e_h, g1 = nx, ne_h, g1n
    return (xc, jnp.concatenate(e_h, axis=0))


# speedup vs baseline: 5.1267x; 1.0064x over previous
import functools

import jax
import jax.numpy as jnp
from jax import lax
from jax.experimental import pallas as pl
from jax.experimental.pallas import tpu as pltpu
from jax.experimental.pallas import tpu_sc as plsc

N = 10000
E = 320000
D = 128

NC = 2
NS = 16
NW = NC * NS
EPW = E // NW
CHUNK = 40
NCH = EPW // CHUNK
NPAD = 10240
RPT = NPAD // NS

def _sc_mesh():
    return plsc.VectorSubcoreMesh(core_axis_name="c", subcore_axis_name="s",
                                  num_cores=NC, num_subcores=NS)


GRP = 5
GROUP = GRP * CHUNK
EH = E // 2


@functools.cache
def _gather_kernel(ne):
    epw = ne // NW
    nch = epw // CHUNK
    ng = nch // GRP

    @functools.partial(
        pl.kernel,
        out_type=jax.ShapeDtypeStruct((ne, D), jnp.float32),
        mesh=_sc_mesh(),
        scratch_types=[
            pltpu.VMEM((nch, CHUNK), jnp.int32),
            pltpu.VMEM((nch, CHUNK), jnp.int32),
            pltpu.VMEM((3, GROUP, D), jnp.float32),
            pltpu.SemaphoreType.DMA,
            pltpu.SemaphoreType.DMA,
            pltpu.SemaphoreType.DMA,
        ],
    )
    def k(pa_hbm, pb_hbm, ia_hbm, ib_hbm, out_hbm, ia_v, ib_v, buf, sa, sb, so):
        wid = lax.axis_index("c") * NS + lax.axis_index("s")
        base = wid * epw
        pltpu.sync_copy(ia_hbm.at[wid], ia_v)
        pltpu.sync_copy(ib_hbm.at[wid], ib_v)

        def mk_a(g, r, i):
            return pltpu.make_async_copy(
                pa_hbm.at[ia_v.at[g * GRP + i]],
                buf.at[r, pl.ds(i * CHUNK, CHUNK)], sa)

        def mk_b(g, r, i):
            return pltpu.make_async_copy(
                pb_hbm.at[ib_v.at[g * GRP + i]],
                buf.at[r, pl.ds(i * CHUNK, CHUNK)], sb)

        def mk_o(g, r):
            return pltpu.make_async_copy(
                buf.at[r], out_hbm.at[pl.ds(base + g * GROUP, GROUP)], so)

        def issue_a(g, r):
            for i in range(GRP):
                mk_a(g, r, i).start()

        issue_a(0, 0)
        issue_a(1, 1)

        def body(g, carry):
            r = g % 3
            for i in range(GRP):
                mk_a(g, r, i).wait()
            for i in range(GRP):
                mk_b(g, r, i).start(add=True)

            @pl.when(g + 2 < ng)
            def _():
                @pl.when(g >= 1)
                def _():
                    mk_o(g - 1, (g - 1) % 3).wait()
                issue_a(g + 2, (g + 2) % 3)

            for i in range(GRP):
                mk_b(g, r, i).wait()
            mk_o(g, r).start()

            return carry

        lax.fori_loop(0, ng, body, 0)
        mk_o(ng - 3, (ng - 3) % 3).wait()
        mk_o(ng - 2, (ng - 2) % 3).wait()
        mk_o(ng - 1, (ng - 1) % 3).wait()

    return k


def _sc_gather_sum(pa, pb, ia, ib):
    return _gather_kernel(ia.shape[0] * ia.shape[1] * ia.shape[2])(pa, pb, ia, ib)


@functools.cache
def _scatter_kernel(ne):
    epw = ne // NW
    sng = epw // CHUNK

    @functools.partial(
        pl.kernel,
        out_type=jax.ShapeDtypeStruct((NC, NPAD, D), jnp.float32),
        mesh=_sc_mesh(),
        scratch_types=[
            pltpu.VMEM((3, CHUNK), jnp.int32),
            pltpu.VMEM((3, CHUNK, D), jnp.float32),
            pltpu.VMEM_SHARED((NPAD, D), jnp.float32),
            pltpu.SemaphoreType.DMA,
            pltpu.SemaphoreType.DMA,
            pltpu.SemaphoreType.DMA,
        ],
    )
    def k(m_hbm, idx_hbm, zeros_hbm, out_hbm, idx_v, buf, acc_sh, si, sx, ss):
        cid = lax.axis_index("c")
        sid = lax.axis_index("s")
        wid = cid * NS + sid
        base = wid * epw
        pltpu.sync_copy(zeros_hbm.at[pl.ds(sid * RPT, RPT)],
                        acc_sh.at[pl.ds(sid * RPT, RPT)])
        plsc.subcore_barrier()

        def mk_i(g, r):
            return pltpu.make_async_copy(
                m_hbm.at[pl.ds(base + g * CHUNK, CHUNK)], buf.at[r], si)

        def mk_x(g, r):
            return pltpu.make_async_copy(idx_hbm.at[wid, g], idx_v.at[r], sx)

        def mk_s(g, r):
            return pltpu.make_async_copy(buf.at[r], acc_sh.at[idx_v.at[r]], ss)

        for gg in range(2):
            mk_i(gg, gg).start()
            mk_x(gg, gg).start()

        def body(g, carry):
            r = g % 3
            mk_i(g, r).wait()
            mk_x(g, r).wait()
            mk_s(g, r).start(add=True)

            @pl.when(g + 2 < sng)
            def _():
                @pl.when(g >= 1)
                def _():
                    mk_s(g - 1, (g - 1) % 3).wait()
                mk_i(g + 2, (g + 2) % 3).start()
                mk_x(g + 2, (g + 2) % 3).start()

            return carry

        lax.fori_loop(0, sng, body, 0)
        mk_s(sng - 3, (sng - 3) % 3).wait()
        mk_s(sng - 2, (sng - 2) % 3).wait()
        mk_s(sng - 1, (sng - 1) % 3).wait()
        plsc.subcore_barrier()
        pltpu.sync_copy(acc_sh.at[pl.ds(sid * RPT, RPT)],
                        out_hbm.at[cid, pl.ds(sid * RPT, RPT)])

    return k


def _sc_scatter_add(m, idx, zeros):
    return _scatter_kernel(m.shape[0])(m, idx, zeros)


def _dot(a, b):
    return jnp.dot(a.astype(jnp.bfloat16), b.astype(jnp.bfloat16),
                   preferred_element_type=jnp.float32)


def _mlp_tail(h, w1, b1, w2, b2, gam, bet):
    h = jnp.maximum(_dot(h, w1) + b1, 0.0)
    h = _dot(h, w2) + b2
    mu = jnp.mean(h, axis=-1, keepdims=True)
    var = jnp.mean((h - mu) ** 2, axis=-1, keepdims=True)
    h = (h - mu) * lax.rsqrt(var + 1e-5)
    return h * gam + bet


def _edge_mlp_body(g_ref, e_ref, wc, b0, w1, b1, w2, b2, gam, bet, o_ref):
    h = jnp.maximum(g_ref[...] + _dot(e_ref[...], wc[...]) + b0[...], 0.0)
    o_ref[...] = _mlp_tail(h, w1[...], b1[...], w2[...], b2[...], gam[...], bet[...])


def _u_mlp_body(g_ref, e_ref, wc, b0, w1, b1, w2, b2, gam, bet, o_ref):
    ev = e_ref[...]
    h = jnp.maximum(g_ref[...] + _dot(ev, wc[...]) + b0[...], 0.0)
    o_ref[...] = _mlp_tail(h, w1[...], b1[...], w2[...], b2[...], gam[...], bet[...]) + ev


def _prep_body(x_ref, wa, wb, p_ref, q_ref):
    xv = x_ref[...]
    p_ref[...] = _dot(xv, wa[...])
    q_ref[...] = _dot(xv, wb[...])


def _node_body(a0_ref, a1_ref, a2_ref, a3_ref, x_ref, w0a, w0b, b0, w1, b1,
               w2, b2, gam, bet, u0a, u0b, ea, eb,
               nx_ref, pp_ref, qq_ref, pn_ref, qn_ref):
    xv = x_ref[...]
    aggr = (a0_ref[...] + a1_ref[...]) + (a2_ref[...] + a3_ref[...])
    h = jnp.maximum(_dot(aggr, w0a[...]) + _dot(xv, w0b[...]) + b0[...], 0.0)
    xu = _mlp_tail(h, w1[...], b1[...], w2[...], b2[...], gam[...], bet[...])
    nx = xu + xv
    nx_ref[...] = nx
    pp_ref[...] = _dot(xu, u0a[...])
    qq_ref[...] = _dot(xu, u0b[...])
    pn_ref[...] = _dot(nx, ea[...])
    qn_ref[...] = _dot(nx, eb[...])


BE = 4000
BN = 1000

_w_spec = pl.BlockSpec((D, D), lambda i: (0, 0))
_b_spec = pl.BlockSpec((1, D), lambda i: (0, 0))


def _rows_spec(rows):
    return pl.BlockSpec((rows, D), lambda i: (i, 0))


def _edge_mlp(body, g, e, wc, b0, w1, b1, w2, b2, gam, bet):
    ne = g.shape[0]
    return pl.pallas_call(
        body,
        grid=(ne // BE,),
        in_specs=[_rows_spec(BE), _rows_spec(BE),
                  _w_spec, _b_spec, _w_spec, _b_spec, _w_spec, _b_spec,
                  _b_spec, _b_spec],
        out_specs=_rows_spec(BE),
        out_shape=jax.ShapeDtypeStruct((ne, D), jnp.float32),
    )(g, e, wc, b0, w1, b1, w2, b2, gam, bet)


def _prep(x, wa, wb):
    return pl.pallas_call(
        _prep_body,
        grid=(N // BN,),
        in_specs=[_rows_spec(BN), _w_spec, _w_spec],
        out_specs=[_rows_spec(BN), _rows_spec(BN)],
        out_shape=[jax.ShapeDtypeStruct((N, D), jnp.float32)] * 2,
    )(x, wa, wb)


def _node_mlp(a0, a1, a2, a3, x, w0a, w0b, b0, w1, b1, w2, b2, gam, bet,
              u0a, u0b, ea, eb):
    return pl.pallas_call(
        _node_body,
        grid=(N // BN,),
        in_specs=[_rows_spec(BN)] * 5
        + [_w_spec, _w_spec, _b_spec, _w_spec, _b_spec, _w_spec, _b_spec,
           _b_spec, _b_spec, _w_spec, _w_spec, _w_spec, _w_spec],
        out_specs=[_rows_spec(BN)] * 5,
        out_shape=[jax.ShapeDtypeStruct((N, D), jnp.float32)] * 5,
    )(a0, a1, a2, a3, x, w0a, w0b, b0, w1, b1, w2, b2, gam, bet, u0a, u0b, ea, eb)


def kernel(x, edge_index, edge_features,
           eW0, eb0, eW1, eb1, eW2, eb2, eg, ebt,
           nW0, nb0, nW1, nb1, nW2, nb2, ng, nbt,
           uW0, ub0, uW1, ub1, uW2, ub2, ug, ubt):
    hch = EH // (NW * CHUNK)
    src_h = [edge_index[0][h * EH:(h + 1) * EH].reshape(NW, hch, CHUNK)
             for h in range(2)]
    tgt_h = [edge_index[1][h * EH:(h + 1) * EH].reshape(NW, hch, CHUNK)
             for h in range(2)]
    e_h = [edge_features[:EH], edge_features[EH:]]
    zeros = jnp.zeros((NPAD, D), jnp.float32)

    def b(v):
        return v.reshape(1, D)

    xc = x
    p, q = _prep(xc, eW0[0][:D], eW0[0][D:2 * D])
    g1 = [_sc_gather_sum(p, q, tgt_h[0], src_h[0]),
          _sc_gather_sum(p, q, tgt_h[1], src_h[1])]
    for s in range(10):
        eW0s, nW0s, uW0s = eW0[s], nW0[s], uW0[s]
        eW0n = eW0[s + 1] if s < 9 else eW0s
        parts = []
        for h in range(2):
            m = _edge_mlp(_edge_mlp_body, g1[h], e_h[h], eW0s[2 * D:],
                          b(eb0[s]), eW1[s], b(eb1[s]), eW2[s], b(eb2[s]),
                          b(eg[s]), b(ebt[s]))
            parts.append(_sc_scatter_add(m, tgt_h[h], zeros))
        nx, pp, qq, pn, qn = _node_mlp(
            parts[0][0], parts[0][1], parts[1][0], parts[1][1], xc,
            nW0s[:D], nW0s[D:], b(nb0[s]), nW1[s], b(nb1[s]), nW2[s],
            b(nb2[s]), b(ng[s]), b(nbt[s]),
            uW0s[:D], uW0s[D:2 * D], eW0n[:D], eW0n[D:2 * D])
        ne_h = [None, None]
        g1n = [None, None]
        for h in range(2):
            g2 = _sc_gather_sum(pp, qq, src_h[h], tgt_h[h])
            if s < 9:
                g1n[h] = _sc_gather_sum(pn, qn, tgt_h[h], src_h[h])
            ne_h[h] = _edge_mlp(_u_mlp_body, g2, e_h[h], uW0s[2 * D:],
                                b(ub0[s]), uW1[s], b(ub1[s]), uW2[s],
                                b(ub2[s]), b(ug[s]), b(ubt[s]))
        xc, e_h, g1 = nx, ne_h, g1n
    return (xc, jnp.concatenate(e_h, axis=0))
